# baseline (speedup 1.0000x reference)
import functools

import jax
import jax.numpy as jnp
from jax import lax
from jax.experimental import pallas as pl
from jax.experimental.pallas import tpu as pltpu
from jax.experimental.pallas import tpu_sc as plsc

TOPK = 500
NBOX = 20000
NCLS = 80
CH = 256
CROP = 14
NW = 32
BPW = 16
NPAD = NW * BPW
LEVELS = ((64, 64, 16), (32, 32, 9), (16, 16, 6))


def _scores_body(cls_ref, out_ref):
    out_ref[:] = jnp.max(cls_ref[:], axis=1)


def _i16(v):
    return jnp.full((16,), v, jnp.int32)


def _f16(v):
    return jnp.full((16,), v, jnp.float32)


def _roi_body(idx_hbm, seltab_hbm, p3_hbm, p4_hbm, p5_hbm,
              consts_hbm, selout_hbm, rois_hbm,
              idx_v, bbuf, consts_v, rect, ridx, obuf, iscr, fscr,
              sem_in, sem_o0, sem_o1):
    wid = lax.axis_index("s") * 2 + lax.axis_index("c")
    base = wid * BPW
    pltpu.sync_copy(idx_hbm.at[pl.ds(base, BPW)], idx_v)
    pltpu.sync_copy(consts_hbm, consts_v)
    pltpu.async_copy(seltab_hbm.at[idx_v], bbuf, sem_in).wait()
    pltpu.sync_copy(bbuf, selout_hbm.at[pl.ds(base, BPW)])

    cvals = consts_v[:]
    inv_hf = cvals[0]
    inv_wf = cvals[1]
    lane = jax.lax.iota(jnp.int32, 16)
    rcf = jnp.minimum(lane, 13).astype(jnp.float32)
    tables = (p3_hbm, p4_hbm, p5_hbm)

    def box_body(i, carry):
        g = base + i

        @pl.when(g < TOPK)
        def _():
            bval = plsc.load_gather(bbuf, [_i16(i), lane])
            x1 = bval[0] * inv_wf
            y1 = bval[1] * inv_hf
            x2 = bval[2] * inv_wf
            y2 = bval[3] * inv_hf
            sems = (sem_o0, sem_o1)
            pending = [None, None]
            for lvl, (hl, wl, rw) in enumerate(LEVELS):
                hl1 = float(hl - 1)
                wl1 = float(wl - 1)
                hs = (y2 - y1) * (hl1 / 13.0)
                ws = (x2 - x1) * (wl1 / 13.0)
                y1s = y1 * hl1
                x1s = x1 * wl1
                in_y = _f16(y1s) + rcf * _f16(hs)
                in_x = _f16(x1s) + rcf * _f16(ws)
                ti = in_y.astype(jnp.int32)
                li = in_x.astype(jnp.int32)
                ylerp = in_y - ti.astype(jnp.float32)
                xlerp = in_x - li.astype(jnp.float32)
                bi = jnp.minimum(ti + (ylerp > 0.0).astype(jnp.int32), hl - 1)
                ri = jnp.minimum(li + (xlerp > 0.0).astype(jnp.int32), wl - 1)
                y0 = jnp.minimum(ti[0], hl - rw)
                x0 = jnp.minimum(li[0], wl - rw)
                iscr[0, :] = jnp.clip(ti - y0, 0, 15) * 16
                iscr[1, :] = jnp.clip(bi - y0, 0, 15) * 16
                iscr[2, :] = jnp.clip(li - x0, 0, 15)
                iscr[3, :] = jnp.clip(ri - x0, 0, 15)
                fscr[0, :] = ylerp
                fscr[1, :] = xlerp
                tbl = tables[lvl]
                nmax = hl * wl - 1
                for rr in range(rw):
                    ridx[rr, :] = jnp.minimum(
                        _i16((y0 + rr) * wl + x0) + lane, nmax)
                gathers = [
                    pltpu.async_copy(tbl.at[ridx.at[rr]],
                                     rect.at[pl.ds(rr * 16, 16)], sem_in)
                    for rr in range(rw)
                ]
                for gth in gathers:
                    gth.wait()
                for half in range(2):
                    if pending[half] is not None:
                        pending[half].wait()
                        pending[half] = None

                    @plsc.parallel_loop(0, 7)
                    def row_body(rloc, half=half):
                        rrv = _i16(half * 7 + rloc)
                        rrt = plsc.load_gather(iscr, [_i16(0), rrv])
                        rrb = plsc.load_gather(iscr, [_i16(1), rrv])
                        ylv = plsc.load_gather(fscr, [_i16(0), rrv])
                        rv = _i16(rloc)

                        @plsc.parallel_loop(0, CROP, unroll=4)
                        def px_body(cc, rrt=rrt, rrb=rrb, ylv=ylv,
                                    rv=rv, half=half):
                            ccv = _i16(cc)
                            ccl = plsc.load_gather(iscr, [_i16(2), ccv])
                            ccr = plsc.load_gather(iscr, [_i16(3), ccv])
                            xlv = plsc.load_gather(fscr, [_i16(1), ccv])
                            cv = _i16(cc)
                            ptl = rrt + ccl
                            ptr = rrt + ccr
                            pbl = rrb + ccl
                            pbr = rrb + ccr
                            for ck in range(CH // 16):
                                cvec = lane + (16 * ck)
                                tl = plsc.load_gather(rect, [ptl, cvec])
                                tr = plsc.load_gather(rect, [ptr, cvec])
                                bl = plsc.load_gather(rect, [pbl, cvec])
                                br = plsc.load_gather(rect, [pbr, cvec])
                                top = tl + (tr - tl) * xlv
                                bot = bl + (br - bl) * xlv
                                res = top + (bot - top) * ylv
                                plsc.store_scatter(
                                    obuf.at[half], [rv, cv, cvec], res)
                    pending[half] = pltpu.async_copy(
                        obuf.at[half],
                        rois_hbm.at[g, pl.ds(half * 7, 7), :,
                                    pl.ds(lvl * CH, CH)],
                        sems[half])
            for half in range(2):
                if pending[half] is not None:
                    pending[half].wait()

        return carry

    lax.fori_loop(0, BPW, box_body, 0)


@functools.partial(
    pl.kernel,
    mesh=plsc.VectorSubcoreMesh(core_axis_name="c", subcore_axis_name="s"),
    compiler_params=pltpu.CompilerParams(
        needs_layout_passes=False, use_tc_tiling_on_sc=False),
    out_type=(
        jax.ShapeDtypeStruct((NPAD, 128), jnp.float32),
        jax.ShapeDtypeStruct((TOPK, CROP, CROP, 3 * CH), jnp.float32),
    ),
    scratch_types=[
        pltpu.VMEM((BPW,), jnp.int32),
        pltpu.VMEM((BPW, 128), jnp.float32),
        pltpu.VMEM((16,), jnp.float32),
        pltpu.VMEM((256, CH), jnp.float32),
        pltpu.VMEM((16, 16), jnp.int32),
        pltpu.VMEM((2, 7, CROP, CH), jnp.float32),
        pltpu.VMEM((4, 16), jnp.int32),
        pltpu.VMEM((2, 16), jnp.float32),
        pltpu.SemaphoreType.DMA,
        pltpu.SemaphoreType.DMA,
        pltpu.SemaphoreType.DMA,
    ],
)
def _roi_sc(*refs):
    _roi_body(*refs)


def kernel(image_shape, boxes, classification, fpn_p3, fpn_p4, fpn_p5):
    cls_all = classification[0]
    scores = pl.pallas_call(
        _scores_body,
        out_shape=jax.ShapeDtypeStruct((cls_all.shape[0],), jnp.float32),
    )(cls_all)
    _, idx = lax.top_k(scores, TOPK)
    idxp = jnp.zeros((NPAD,), jnp.int32).at[:TOPK].set(idx)
    seltab = jnp.concatenate(
        [boxes[0], cls_all,
         jnp.zeros((NBOX, 128 - 4 - NCLS), jnp.float32)], axis=1)
    consts = jnp.zeros((16,), jnp.float32)
    consts = consts.at[0].set(1.0 / image_shape[1].astype(jnp.float32))
    consts = consts.at[1].set(1.0 / image_shape[2].astype(jnp.float32))
    p3f = fpn_p3.reshape(64 * 64, CH)
    p4f = fpn_p4.reshape(32 * 32, CH)
    p5f = fpn_p5.reshape(16 * 16, CH)
    selout, rois = _roi_sc(idxp, seltab, p3f, p4f, p5f, consts)
    return (selout[:TOPK, 0:4][None], selout[:TOPK, 4:4 + NCLS][None],
            rois[None])


# speedup vs baseline: 1.3219x; 1.3219x over previous
import functools

import jax
import jax.numpy as jnp
from jax import lax
from jax.experimental import pallas as pl
from jax.experimental.pallas import tpu as pltpu
from jax.experimental.pallas import tpu_sc as plsc

TOPK = 500
NBOX = 20000
NCLS = 80
CH = 256
CROP = 14
NW = 32
BPW = 16
NPAD = NW * BPW
LEVELS = ((64, 64, 16), (32, 32, 9), (16, 16, 6))


def _scores_body(cls_ref, out_ref):
    out_ref[:] = jnp.max(cls_ref[:], axis=1)


def _i16(v):
    return jnp.full((16,), v, jnp.int32)


def _f16(v):
    return jnp.full((16,), v, jnp.float32)


def _roi_body(idx_hbm, seltab_hbm, p3_hbm, p4_hbm, p5_hbm,
              consts_hbm, selout_hbm, rois_hbm,
              idx_v, bbuf, consts_v, rect, ridx, obuf, iscr, fscr,
              sem_in, sem_o0, sem_o1):
    wid = lax.axis_index("s") * 2 + lax.axis_index("c")
    base = wid * BPW
    pltpu.sync_copy(idx_hbm.at[pl.ds(base, BPW)], idx_v)
    pltpu.sync_copy(consts_hbm, consts_v)
    pltpu.async_copy(seltab_hbm.at[idx_v], bbuf, sem_in).wait()
    pltpu.sync_copy(bbuf, selout_hbm.at[pl.ds(base, BPW)])

    cvals = consts_v[:]
    inv_hf = cvals[0]
    inv_wf = cvals[1]
    lane = jax.lax.iota(jnp.int32, 16)
    rcf = jnp.minimum(lane, 13).astype(jnp.float32)
    tables = (p3_hbm, p4_hbm, p5_hbm)

    def box_body(i, carry):
        g = base + i

        @pl.when(g < TOPK)
        def _():
            bval = plsc.load_gather(bbuf, [_i16(i), lane])
            x1 = bval[0] * inv_wf
            y1 = bval[1] * inv_hf
            x2 = bval[2] * inv_wf
            y2 = bval[3] * inv_hf
            sems = (sem_o0, sem_o1)
            pending = [None, None]
            for lvl, (hl, wl, rw) in enumerate(LEVELS):
                hl1 = float(hl - 1)
                wl1 = float(wl - 1)
                hs = (y2 - y1) * (hl1 / 13.0)
                ws = (x2 - x1) * (wl1 / 13.0)
                y1s = y1 * hl1
                x1s = x1 * wl1
                in_y = _f16(y1s) + rcf * _f16(hs)
                in_x = _f16(x1s) + rcf * _f16(ws)
                ti = in_y.astype(jnp.int32)
                li = in_x.astype(jnp.int32)
                ylerp = in_y - ti.astype(jnp.float32)
                xlerp = in_x - li.astype(jnp.float32)
                bi = jnp.minimum(ti + (ylerp > 0.0).astype(jnp.int32), hl - 1)
                ri = jnp.minimum(li + (xlerp > 0.0).astype(jnp.int32), wl - 1)
                y0 = jnp.minimum(ti[0], hl - rw)
                x0 = jnp.minimum(li[0], wl - rw)
                iscr[0, :] = jnp.clip(ti - y0, 0, 15) * 16
                iscr[1, :] = jnp.clip(bi - y0, 0, 15) * 16
                iscr[2, :] = jnp.clip(li - x0, 0, 15)
                iscr[3, :] = jnp.clip(ri - x0, 0, 15)
                fscr[0, :] = ylerp
                fscr[1, :] = xlerp
                tbl = tables[lvl]
                nmax = hl * wl - 1
                for rr in range(rw):
                    ridx[rr, :] = jnp.minimum(
                        _i16((y0 + rr) * wl + x0) + lane, nmax)
                gathers = [
                    pltpu.async_copy(tbl.at[ridx.at[rr]],
                                     rect.at[pl.ds(rr * 16, 16)], sem_in)
                    for rr in range(rw)
                ]
                for gth in gathers:
                    gth.wait()
                for half in range(2):
                    if pending[half] is not None:
                        pending[half].wait()
                        pending[half] = None

                    @plsc.parallel_loop(0, 7)
                    def row_body(rloc, half=half):
                        rrv = _i16(half * 7 + rloc)
                        rrt = plsc.load_gather(iscr, [_i16(0), rrv])
                        rrb = plsc.load_gather(iscr, [_i16(1), rrv])
                        ylv = plsc.load_gather(fscr, [_i16(0), rrv])
                        rv = _i16(rloc)

                        @plsc.parallel_loop(0, CROP, unroll=2)
                        def px_body(cc, rrt=rrt, rrb=rrb, ylv=ylv,
                                    rv=rv, half=half):
                            ccv = _i16(cc)
                            ccl = plsc.load_gather(iscr, [_i16(2), ccv])
                            ccr = plsc.load_gather(iscr, [_i16(3), ccv])
                            xlv = plsc.load_gather(fscr, [_i16(1), ccv])
                            cv = _i16(cc)
                            ptl = rrt + ccl
                            ptr = rrt + ccr
                            pbl = rrb + ccl
                            pbr = rrb + ccr
                            for ck in range(CH // 16):
                                cvec = lane + (16 * ck)
                                tl = plsc.load_gather(rect, [ptl, cvec])
                                tr = plsc.load_gather(rect, [ptr, cvec])
                                bl = plsc.load_gather(rect, [pbl, cvec])
                                br = plsc.load_gather(rect, [pbr, cvec])
                                top = tl + (tr - tl) * xlv
                                bot = bl + (br - bl) * xlv
                                res = top + (bot - top) * ylv
                                plsc.store_scatter(
                                    obuf.at[half], [rv, cv, cvec], res)
                    pending[half] = pltpu.async_copy(
                        obuf.at[half],
                        rois_hbm.at[g, pl.ds(half * 7, 7), :,
                                    pl.ds(lvl * CH, CH)],
                        sems[half])
            for half in range(2):
                if pending[half] is not None:
                    pending[half].wait()

        return carry

    lax.fori_loop(0, BPW, box_body, 0)


@functools.partial(
    pl.kernel,
    mesh=plsc.VectorSubcoreMesh(core_axis_name="c", subcore_axis_name="s"),
    compiler_params=pltpu.CompilerParams(
        needs_layout_passes=False, use_tc_tiling_on_sc=False),
    out_type=(
        jax.ShapeDtypeStruct((NPAD, 128), jnp.float32),
        jax.ShapeDtypeStruct((TOPK, CROP, CROP, 3 * CH), jnp.float32),
    ),
    scratch_types=[
        pltpu.VMEM((BPW,), jnp.int32),
        pltpu.VMEM((BPW, 128), jnp.float32),
        pltpu.VMEM((16,), jnp.float32),
        pltpu.VMEM((256, CH), jnp.float32),
        pltpu.VMEM((16, 16), jnp.int32),
        pltpu.VMEM((2, 7, CROP, CH), jnp.float32),
        pltpu.VMEM((4, 16), jnp.int32),
        pltpu.VMEM((2, 16), jnp.float32),
        pltpu.SemaphoreType.DMA,
        pltpu.SemaphoreType.DMA,
        pltpu.SemaphoreType.DMA,
    ],
)
def _roi_sc(*refs):
    _roi_body(*refs)


def kernel(image_shape, boxes, classification, fpn_p3, fpn_p4, fpn_p5):
    cls_all = classification[0]
    scores = pl.pallas_call(
        _scores_body,
        out_shape=jax.ShapeDtypeStruct((cls_all.shape[0],), jnp.float32),
    )(cls_all)
    _, idx = lax.top_k(scores, TOPK)
    idxp = jnp.zeros((NPAD,), jnp.int32).at[:TOPK].set(idx)
    seltab = jnp.concatenate(
        [boxes[0], cls_all,
         jnp.zeros((NBOX, 128 - 4 - NCLS), jnp.float32)], axis=1)
    consts = jnp.zeros((16,), jnp.float32)
    consts = consts.at[0].set(1.0 / image_shape[1].astype(jnp.float32))
    consts = consts.at[1].set(1.0 / image_shape[2].astype(jnp.float32))
    p3f = fpn_p3.reshape(64 * 64, CH)
    p4f = fpn_p4.reshape(32 * 32, CH)
    p5f = fpn_p5.reshape(16 * 16, CH)
    selout, rois = _roi_sc(idxp, seltab, p3f, p4f, p5f, consts)
    return (selout[:TOPK, 0:4][None], selout[:TOPK, 4:4 + NCLS][None],
            rois[None])


# speedup vs baseline: 1.3429x; 1.0160x over previous
import functools

import jax
import jax.numpy as jnp
from jax import lax
from jax.experimental import pallas as pl
from jax.experimental.pallas import tpu as pltpu
from jax.experimental.pallas import tpu_sc as plsc

TOPK = 500
NBOX = 20000
NCLS = 80
CH = 256
CROP = 14
NW = 32
BPW = 16
NPAD = NW * BPW
LEVELS = ((64, 64, 16), (32, 32, 9), (16, 16, 6))


def _scores_body(cls_ref, out_ref):
    out_ref[:] = jnp.max(cls_ref[:], axis=1)


def _i16(v):
    return jnp.full((16,), v, jnp.int32)


def _f16(v):
    return jnp.full((16,), v, jnp.float32)


def _roi_body(idx_hbm, seltab_hbm, p3_hbm, p4_hbm, p5_hbm,
              consts_hbm, selout_hbm, rois_hbm,
              idx_v, bbuf, consts_v, rect, ridx, obuf, iscr, fscr,
              sem_in, sem_o0, sem_o1):
    wid = lax.axis_index("s") * 2 + lax.axis_index("c")
    base = wid * BPW
    pltpu.sync_copy(idx_hbm.at[pl.ds(base, BPW)], idx_v)
    pltpu.sync_copy(consts_hbm, consts_v)
    pltpu.async_copy(seltab_hbm.at[idx_v], bbuf, sem_in).wait()
    pltpu.sync_copy(bbuf, selout_hbm.at[pl.ds(base, BPW)])

    cvals = consts_v[:]
    inv_hf = cvals[0]
    inv_wf = cvals[1]
    lane = jax.lax.iota(jnp.int32, 16)
    rcf = jnp.minimum(lane, 13).astype(jnp.float32)
    tables = (p3_hbm, p4_hbm, p5_hbm)

    def box_body(i, carry):
        g = base + i

        @pl.when(g < TOPK)
        def _():
            bval = plsc.load_gather(bbuf, [_i16(i), lane])
            x1 = bval[0] * inv_wf
            y1 = bval[1] * inv_hf
            x2 = bval[2] * inv_wf
            y2 = bval[3] * inv_hf
            sems = (sem_o0, sem_o1)
            pending = [None, None]
            for lvl, (hl, wl, rw) in enumerate(LEVELS):
                hl1 = float(hl - 1)
                wl1 = float(wl - 1)
                hs = (y2 - y1) * (hl1 / 13.0)
                ws = (x2 - x1) * (wl1 / 13.0)
                y1s = y1 * hl1
                x1s = x1 * wl1
                in_y = _f16(y1s) + rcf * _f16(hs)
                in_x = _f16(x1s) + rcf * _f16(ws)
                ti = in_y.astype(jnp.int32)
                li = in_x.astype(jnp.int32)
                ylerp = in_y - ti.astype(jnp.float32)
                xlerp = in_x - li.astype(jnp.float32)
                bi = jnp.minimum(ti + (ylerp > 0.0).astype(jnp.int32), hl - 1)
                ri = jnp.minimum(li + (xlerp > 0.0).astype(jnp.int32), wl - 1)
                y0 = jnp.minimum(ti[0], hl - rw)
                x0 = jnp.minimum(li[0], wl - rw)
                iscr[0, :] = jnp.clip(ti - y0, 0, 15) * 16
                iscr[1, :] = jnp.clip(bi - y0, 0, 15) * 16
                iscr[2, :] = jnp.clip(li - x0, 0, 15)
                iscr[3, :] = jnp.clip(ri - x0, 0, 15)
                fscr[0, :] = ylerp
                fscr[1, :] = xlerp
                tbl = tables[lvl]
                nmax = hl * wl - 1
                for rr in range(rw):
                    ridx[rr, :] = jnp.minimum(
                        _i16((y0 + rr) * wl + x0) + lane, nmax)
                gathers = [
                    pltpu.async_copy(tbl.at[ridx.at[rr]],
                                     rect.at[pl.ds(rr * 16, 16)], sem_in)
                    for rr in range(rw)
                ]
                for gth in gathers:
                    gth.wait()
                for half in range(2):
                    if pending[half] is not None:
                        pending[half].wait()
                        pending[half] = None
                    elif lvl == 0:
                        @pl.when(i > 0)
                        def _(half=half):
                            pltpu.make_async_copy(
                                rois_hbm.at[0, pl.ds(half * 7, 7), :,
                                            pl.ds(0, CH)],
                                obuf.at[half], sems[half]).wait()

                    @plsc.parallel_loop(0, 7)
                    def row_body(rloc, half=half):
                        rrv = _i16(half * 7 + rloc)
                        rrt = plsc.load_gather(iscr, [_i16(0), rrv])
                        rrb = plsc.load_gather(iscr, [_i16(1), rrv])
                        ylv = plsc.load_gather(fscr, [_i16(0), rrv])
                        rv = _i16(rloc)

                        @plsc.parallel_loop(0, CROP, unroll=2)
                        def px_body(cc, rrt=rrt, rrb=rrb, ylv=ylv,
                                    rv=rv, half=half):
                            ccv = _i16(cc)
                            ccl = plsc.load_gather(iscr, [_i16(2), ccv])
                            ccr = plsc.load_gather(iscr, [_i16(3), ccv])
                            xlv = plsc.load_gather(fscr, [_i16(1), ccv])
                            cv = _i16(cc)
                            ptl = rrt + ccl
                            ptr = rrt + ccr
                            pbl = rrb + ccl
                            pbr = rrb + ccr
                            for ck in range(CH // 16):
                                cvec = lane + (16 * ck)
                                tl = plsc.load_gather(rect, [ptl, cvec])
                                tr = plsc.load_gather(rect, [ptr, cvec])
                                bl = plsc.load_gather(rect, [pbl, cvec])
                                br = plsc.load_gather(rect, [pbr, cvec])
                                top = tl + (tr - tl) * xlv
                                bot = bl + (br - bl) * xlv
                                res = top + (bot - top) * ylv
                                plsc.store_scatter(
                                    obuf.at[half], [rv, cv, cvec], res)
                    pending[half] = pltpu.async_copy(
                        obuf.at[half],
                        rois_hbm.at[g, pl.ds(half * 7, 7), :,
                                    pl.ds(lvl * CH, CH)],
                        sems[half])
        return carry

    lax.fori_loop(0, BPW, box_body, 0)
    for half, sem in enumerate((sem_o0, sem_o1)):
        pltpu.make_async_copy(
            rois_hbm.at[0, pl.ds(half * 7, 7), :, pl.ds(0, CH)],
            obuf.at[half], sem).wait()


@functools.partial(
    pl.kernel,
    mesh=plsc.VectorSubcoreMesh(core_axis_name="c", subcore_axis_name="s"),
    compiler_params=pltpu.CompilerParams(
        needs_layout_passes=False, use_tc_tiling_on_sc=False),
    out_type=(
        jax.ShapeDtypeStruct((NPAD, 128), jnp.float32),
        jax.ShapeDtypeStruct((TOPK, CROP, CROP, 3 * CH), jnp.float32),
    ),
    scratch_types=[
        pltpu.VMEM((BPW,), jnp.int32),
        pltpu.VMEM((BPW, 128), jnp.float32),
        pltpu.VMEM((16,), jnp.float32),
        pltpu.VMEM((256, CH), jnp.float32),
        pltpu.VMEM((16, 16), jnp.int32),
        pltpu.VMEM((2, 7, CROP, CH), jnp.float32),
        pltpu.VMEM((4, 16), jnp.int32),
        pltpu.VMEM((2, 16), jnp.float32),
        pltpu.SemaphoreType.DMA,
        pltpu.SemaphoreType.DMA,
        pltpu.SemaphoreType.DMA,
    ],
)
def _roi_sc(*refs):
    _roi_body(*refs)


def kernel(image_shape, boxes, classification, fpn_p3, fpn_p4, fpn_p5):
    cls_all = classification[0]
    scores = pl.pallas_call(
        _scores_body,
        out_shape=jax.ShapeDtypeStruct((cls_all.shape[0],), jnp.float32),
    )(cls_all)
    _, idx = lax.top_k(scores, TOPK)
    idxp = jnp.zeros((NPAD,), jnp.int32).at[:TOPK].set(idx)
    seltab = jnp.concatenate(
        [boxes[0], cls_all,
         jnp.zeros((NBOX, 128 - 4 - NCLS), jnp.float32)], axis=1)
    consts = jnp.zeros((16,), jnp.float32)
    consts = consts.at[0].set(1.0 / image_shape[1].astype(jnp.float32))
    consts = consts.at[1].set(1.0 / image_shape[2].astype(jnp.float32))
    p3f = fpn_p3.reshape(64 * 64, CH)
    p4f = fpn_p4.reshape(32 * 32, CH)
    p5f = fpn_p5.reshape(16 * 16, CH)
    selout, rois = _roi_sc(idxp, seltab, p3f, p4f, p5f, consts)
    return (selout[:TOPK, 0:4][None], selout[:TOPK, 4:4 + NCLS][None],
            rois[None])
